# baseline (speedup 1.0000x reference)
import functools

import jax
import jax.numpy as jnp
from jax.experimental import pallas as pl


def _layer_body(adj_ref, v_ref, w_ref, b_ref, o_ref, *, relu: bool):
    t = jnp.dot(adj_ref[...].astype(jnp.bfloat16), v_ref[...],
                preferred_element_type=jnp.float32)
    o = jnp.dot(t, w_ref[...], preferred_element_type=jnp.float32) + b_ref[...]
    if relu:
        o = jnp.maximum(o, 0.0)
    o_ref[...] = o.astype(o_ref.dtype)


def _gcn_layer(adj, v, w, b, *, relu: bool, block_rows: int, out_dtype):
    n, k = adj.shape
    d = w.shape[1]
    return pl.pallas_call(
        functools.partial(_layer_body, relu=relu),
        grid=(pl.cdiv(n, block_rows),),
        in_specs=[
            pl.BlockSpec((block_rows, k), lambda i: (i, 0)),
            pl.BlockSpec((k, v.shape[1]), lambda i: (0, 0)),
            pl.BlockSpec((v.shape[1], d), lambda i: (0, 0)),
            pl.BlockSpec((1, d), lambda i: (0, 0)),
        ],
        out_specs=pl.BlockSpec((block_rows, d), lambda i: (i, 0)),
        out_shape=jax.ShapeDtypeStruct((n, d), out_dtype),
    )(adj, v, w, b)


def kernel(adj, x, W1, b1, W2, b2):
    h = _gcn_layer(adj, x.astype(jnp.bfloat16), W1, b1.reshape(1, -1),
                   relu=True, block_rows=560, out_dtype=jnp.bfloat16)
    out = _gcn_layer(adj, h, W2, b2.reshape(1, -1),
                     relu=False, block_rows=560, out_dtype=jnp.float32)
    return out


# speedup vs baseline: 1.0265x; 1.0265x over previous
import jax
import jax.numpy as jnp
from jax.experimental import pallas as pl
from jax.experimental.pallas import tpu as pltpu

_BI = 400


def _fused_body(adj_ref, x_ref, w1_ref, b1_ref, w2_ref, b2_ref, o_ref, h_ref):
    p = pl.program_id(0)
    i = pl.program_id(1)
    a = adj_ref[...].astype(jnp.bfloat16)

    @pl.when(p == 0)
    def _layer1():
        t = jnp.dot(a, x_ref[...], preferred_element_type=jnp.float32)
        o = jnp.dot(t, w1_ref[...], preferred_element_type=jnp.float32)
        o = jnp.maximum(o + b1_ref[...], 0.0)
        h_ref[pl.ds(i * _BI, _BI), :] = o.astype(jnp.bfloat16)

    @pl.when(p == 1)
    def _layer2():
        t = jnp.dot(a, h_ref[...], preferred_element_type=jnp.float32)
        o_ref[...] = jnp.dot(t, w2_ref[...],
                             preferred_element_type=jnp.float32) + b2_ref[...]


def kernel(adj, x, W1, b1, W2, b2):
    n, _ = adj.shape
    d = W2.shape[1]
    return pl.pallas_call(
        _fused_body,
        grid=(2, n // _BI),
        in_specs=[
            pl.BlockSpec((_BI, n), lambda p, i: (i, 0)),
            pl.BlockSpec((n, x.shape[1]), lambda p, i: (0, 0)),
            pl.BlockSpec((x.shape[1], W1.shape[1]), lambda p, i: (0, 0)),
            pl.BlockSpec((1, W1.shape[1]), lambda p, i: (0, 0)),
            pl.BlockSpec((W1.shape[1], d), lambda p, i: (0, 0)),
            pl.BlockSpec((1, d), lambda p, i: (0, 0)),
        ],
        out_specs=pl.BlockSpec((_BI, d), lambda p, i: (i, 0)),
        out_shape=jax.ShapeDtypeStruct((n, d), jnp.float32),
        scratch_shapes=[pltpu.VMEM((n, W1.shape[1]), jnp.bfloat16)],
    )(adj, x.astype(jnp.bfloat16), W1, b1.reshape(1, -1), W2,
      b2.reshape(1, -1))


# speedup vs baseline: 1.0411x; 1.0142x over previous
import jax
import jax.numpy as jnp
from jax.experimental import pallas as pl
from jax.experimental.pallas import tpu as pltpu

_BI = 400


def _fused_body(adj_ref, x_ref, w1_ref, b1_ref, w2_ref, b2_ref, o_ref, h_ref):
    p = pl.program_id(0)
    i = pl.program_id(1)
    a = adj_ref[...].astype(jnp.bfloat16)

    @pl.when(p == 0)
    def _layer1():
        t = jnp.dot(a, x_ref[...].astype(jnp.bfloat16),
                    preferred_element_type=jnp.float32)
        o = jnp.dot(t, w1_ref[...], preferred_element_type=jnp.float32)
        o = jnp.maximum(o + b1_ref[...], 0.0)
        h_ref[pl.ds(i * _BI, _BI), :] = o.astype(jnp.bfloat16)

    @pl.when(p == 1)
    def _layer2():
        t = jnp.dot(a, h_ref[...], preferred_element_type=jnp.float32)
        o_ref[...] = jnp.dot(t, w2_ref[...],
                             preferred_element_type=jnp.float32) + b2_ref[...]


def kernel(adj, x, W1, b1, W2, b2):
    n, _ = adj.shape
    d = W2.shape[1]
    return pl.pallas_call(
        _fused_body,
        grid=(2, n // _BI),
        in_specs=[
            pl.BlockSpec((_BI, n), lambda p, i: (i, 0)),
            pl.BlockSpec((n, x.shape[1]), lambda p, i: (0, 0)),
            pl.BlockSpec((x.shape[1], W1.shape[1]), lambda p, i: (0, 0)),
            pl.BlockSpec((1, W1.shape[1]), lambda p, i: (0, 0)),
            pl.BlockSpec((W1.shape[1], d), lambda p, i: (0, 0)),
            pl.BlockSpec((1, d), lambda p, i: (0, 0)),
        ],
        out_specs=pl.BlockSpec((_BI, d), lambda p, i: (i, 0)),
        out_shape=jax.ShapeDtypeStruct((n, d), jnp.float32),
        scratch_shapes=[pltpu.VMEM((n, W1.shape[1]), jnp.bfloat16)],
    )(adj, x, W1, b1.reshape(1, -1), W2, b2.reshape(1, -1))
